# Initial kernel scaffold; baseline (speedup 1.0000x reference)
import functools

import jax
import jax.numpy as jnp
from jax import lax
from jax.experimental import pallas as pl
from jax.experimental.pallas import tpu as pltpu
from jax.experimental.pallas import tpu_sc as plsc

N = 10000
NP = 10240
E = 320000
EP = 327680
IN_DIM = 128
HID = 128
OUT = 64

NC = 2
NS = 16
NW = NC * NS
EPW = EP // NW
CH = 128
NCHUNK = EPW // CH
RPT = NP // NS


def _zero_rows(ref, nrows, ncols):
    def body(i, _):
        for j in range(ncols // 16):
            ref[i, pl.ds(j * 16, 16)] = jnp.zeros((16,), jnp.float32)
        return 0
    lax.fori_loop(0, nrows, body, 0)


def _make_segsum(D, with_deg):
    mesh = plsc.VectorSubcoreMesh(
        core_axis_name="c", subcore_axis_name="s",
        num_cores=NC, num_subcores=NS)

    out_type = [jax.ShapeDtypeStruct((NC, NP, D), jnp.float32)]
    scratch = [
        pltpu.VMEM((CH,), jnp.int32),
        pltpu.VMEM((CH,), jnp.int32),
        pltpu.VMEM((CH, D), jnp.float32),
        pltpu.VMEM_SHARED((NP, D), jnp.float32),
        pltpu.SemaphoreType.DMA,
    ]
    if with_deg:
        out_type.append(jax.ShapeDtypeStruct((NC, NP, 16), jnp.float32))
        scratch += [
            pltpu.VMEM((CH, 16), jnp.float32),
            pltpu.VMEM_SHARED((NP, 16), jnp.float32),
        ]

    def body(z_hbm, src_hbm, dst_hbm, out_hbm, *rest):
        if with_deg:
            deg_hbm, src_v, dst_v, rows_v, acc_sh, sem, ones_v, deg_sh = rest
        else:
            src_v, dst_v, rows_v, acc_sh, sem = rest
        cid = lax.axis_index("c")
        sid = lax.axis_index("s")
        wid = sid * NC + cid
        row0 = sid * RPT

        _zero_rows(rows_v, CH, D)
        for k in range(RPT // CH):
            pltpu.sync_copy(rows_v, acc_sh.at[pl.ds(row0 + k * CH, CH)])
        if with_deg:
            _zero_rows(ones_v, CH, 16)
            for k in range(RPT // CH):
                pltpu.sync_copy(ones_v, deg_sh.at[pl.ds(row0 + k * CH, CH)])

            def fill_ones(i, _):
                ones_v[i, :] = jnp.ones((16,), jnp.float32)
                return 0
            lax.fori_loop(0, CH, fill_ones, 0)

        plsc.subcore_barrier()

        ebase = wid * EPW

        def chunk(c, _):
            off = ebase + c * CH
            pltpu.sync_copy(src_hbm.at[pl.ds(off, CH)], src_v)
            pltpu.sync_copy(dst_hbm.at[pl.ds(off, CH)], dst_v)
            pltpu.async_copy(z_hbm.at[src_v], rows_v, sem).wait()
            pltpu.sync_copy(rows_v, acc_sh.at[dst_v], add=True)
            if with_deg:
                pltpu.sync_copy(ones_v, deg_sh.at[dst_v], add=True)
            return 0
        lax.fori_loop(0, NCHUNK, chunk, 0)

        plsc.subcore_barrier()

        for k in range(RPT // CH):
            r0 = row0 + k * CH
            pltpu.sync_copy(acc_sh.at[pl.ds(r0, CH)], out_hbm.at[cid, pl.ds(r0, CH)])
            if with_deg:
                pltpu.sync_copy(deg_sh.at[pl.ds(r0, CH)], deg_hbm.at[cid, pl.ds(r0, CH)])

    return pl.kernel(body, out_type=tuple(out_type), mesh=mesh,
                     scratch_types=tuple(scratch))


_segsum_deg_128 = _make_segsum(IN_DIM, with_deg=True)
_segsum_64 = _make_segsum(OUT, with_deg=False)

_TCR = 1024


def _tc_layer1(x, p0, p1, d0, d1, wl1t, b1, wr1t, wl2t, wr2t):
    def body(x_r, p0_r, p1_r, d0_r, d1_r, wl1_r, b1_r, wr1_r, wl2_r, wr2_r,
             z2_r, r2_r):
        invd = 1.0 / jnp.maximum(d0_r[:, 0:1] + d1_r[:, 0:1], 1.0)
        agg = (p0_r[...] + p1_r[...]) * invd
        out1 = (jnp.dot(agg, wl1_r[...], preferred_element_type=jnp.float32)
                + b1_r[...]
                + jnp.dot(x_r[...], wr1_r[...], preferred_element_type=jnp.float32))
        nrm = jnp.sqrt(jnp.sum(out1 * out1, axis=-1, keepdims=True))
        h = jnp.maximum(out1 / jnp.maximum(nrm, 1e-12), 0.0)
        z2_r[...] = jnp.dot(h, wl2_r[...], preferred_element_type=jnp.float32)
        r2_r[...] = jnp.dot(h, wr2_r[...], preferred_element_type=jnp.float32)

    grid = (NP // _TCR,)
    row = lambda i: (i, 0)
    fix = lambda i: (0, 0)
    return pl.pallas_call(
        body,
        grid=grid,
        in_specs=[
            pl.BlockSpec((_TCR, IN_DIM), row),
            pl.BlockSpec((_TCR, IN_DIM), row),
            pl.BlockSpec((_TCR, IN_DIM), row),
            pl.BlockSpec((_TCR, 16), row),
            pl.BlockSpec((_TCR, 16), row),
            pl.BlockSpec((IN_DIM, HID), fix),
            pl.BlockSpec((1, HID), fix),
            pl.BlockSpec((IN_DIM, HID), fix),
            pl.BlockSpec((HID, OUT), fix),
            pl.BlockSpec((HID, OUT), fix),
        ],
        out_specs=[
            pl.BlockSpec((_TCR, OUT), row),
            pl.BlockSpec((_TCR, OUT), row),
        ],
        out_shape=[
            jax.ShapeDtypeStruct((NP, OUT), jnp.float32),
            jax.ShapeDtypeStruct((NP, OUT), jnp.float32),
        ],
    )(x, p0, p1, d0, d1, wl1t, b1, wr1t, wl2t, wr2t)


def _tc_layer2(q0, q1, d0, d1, r2, b2):
    def body(q0_r, q1_r, d0_r, d1_r, r2_r, b2_r, o_r):
        invd = 1.0 / jnp.maximum(d0_r[:, 0:1] + d1_r[:, 0:1], 1.0)
        out2 = (q0_r[...] + q1_r[...]) * invd + b2_r[...] + r2_r[...]
        nrm = jnp.sqrt(jnp.sum(out2 * out2, axis=-1, keepdims=True))
        h2 = jnp.maximum(out2 / jnp.maximum(nrm, 1e-12), 0.0)
        m = jnp.max(h2, axis=-1, keepdims=True)
        e = jnp.exp(h2 - m)
        o_r[...] = (h2 - m) - jnp.log(jnp.sum(e, axis=-1, keepdims=True))

    grid = (NP // _TCR,)
    row = lambda i: (i, 0)
    fix = lambda i: (0, 0)
    return pl.pallas_call(
        body,
        grid=grid,
        in_specs=[
            pl.BlockSpec((_TCR, OUT), row),
            pl.BlockSpec((_TCR, OUT), row),
            pl.BlockSpec((_TCR, 16), row),
            pl.BlockSpec((_TCR, 16), row),
            pl.BlockSpec((_TCR, OUT), row),
            pl.BlockSpec((1, OUT), fix),
        ],
        out_specs=pl.BlockSpec((_TCR, OUT), row),
        out_shape=jax.ShapeDtypeStruct((NP, OUT), jnp.float32),
    )(q0, q1, d0, d1, r2, b2)


def kernel(x, edge_index, W_l1, b_l1, W_r1, W_l2, b_l2, W_r2):
    x = x.astype(jnp.float32)
    src = edge_index[0].astype(jnp.int32)
    dst = edge_index[1].astype(jnp.int32)

    npad = EP - E
    spread = N + (jnp.arange(npad, dtype=jnp.int32) % (NP - N))
    src_p = jnp.concatenate([src, spread])
    dst_p = jnp.concatenate([dst, spread])
    x_p = jnp.pad(x, ((0, NP - N), (0, 0)))

    p, degp = _segsum_deg_128(x_p, src_p, dst_p)

    z2, r2 = _tc_layer1(x_p, p[0], p[1], degp[0], degp[1],
                        W_l1.T, b_l1.reshape(1, HID), W_r1.T,
                        W_l2.T, W_r2.T)

    (q,) = _segsum_64(z2, src_p, dst_p)

    out = _tc_layer2(q[0], q[1], degp[0], degp[1], r2, b_l2.reshape(1, OUT))
    return out[:N]


# speedup vs baseline: 6.0876x; 6.0876x over previous
import jax
import jax.numpy as jnp
from jax import lax
from jax.experimental import pallas as pl
from jax.experimental.pallas import tpu as pltpu
from jax.experimental.pallas import tpu_sc as plsc

N = 10000
NP = 10240
E = 320000
EP = 327680
IN_DIM = 128
HID = 128
OUT = 64

NC = 2
NS = 16
NW = NC * NS
EPW = EP // NW
CH = 128
NCHUNK = EPW // CH
RPT = NP // NS
NSTR = 8


def _zero_rows(ref, nrows, ncols):
    def body(i, _):
        for j in range(ncols // 16):
            ref[i, pl.ds(j * 16, 16)] = jnp.zeros((16,), jnp.float32)
        return 0
    lax.fori_loop(0, nrows, body, 0)


def _make_segsum(D, stage_operand):
    mesh = plsc.VectorSubcoreMesh(
        core_axis_name="c", subcore_axis_name="s",
        num_cores=NC, num_subcores=NS)

    scratch = [
        pltpu.VMEM((CH,), jnp.int32),
        pltpu.VMEM((CH,), jnp.int32),
        pltpu.VMEM((CH, D), jnp.float32),
        pltpu.VMEM_SHARED((2 * NP if stage_operand else NP, D), jnp.float32),
        pltpu.SemaphoreType.DMA,
    ]

    def body(z_hbm, src_hbm, dst_hbm, out_hbm, *rest):
        src_v, dst_v, rows_v, acc_sh, sem = rest
        cid = lax.axis_index("c")
        sid = lax.axis_index("s")
        wid = sid * NC + cid
        row0 = sid * RPT

        if stage_operand:
            for k in range(RPT // CH):
                r0 = row0 + k * CH
                pltpu.sync_copy(z_hbm.at[pl.ds(r0, CH)],
                                acc_sh.at[pl.ds(NP + r0, CH)])
        _zero_rows(rows_v, CH, D)
        for k in range(RPT // CH):
            pltpu.sync_copy(rows_v, acc_sh.at[pl.ds(row0 + k * CH, CH)])

        plsc.subcore_barrier()

        ebase = wid * EPW
        z_src = acc_sh if stage_operand else z_hbm

        def chunk(c, _):
            off = ebase + c * CH
            pltpu.sync_copy(src_hbm.at[pl.ds(off, CH)], src_v)
            pltpu.sync_copy(dst_hbm.at[pl.ds(off, CH)], dst_v)
            pltpu.async_copy(z_src.at[src_v], rows_v, sem).wait()
            pltpu.sync_copy(rows_v, acc_sh.at[dst_v], add=True)
            return 0
        lax.fori_loop(0, NCHUNK, chunk, 0)

        plsc.subcore_barrier()

        for k in range(RPT // CH):
            r0 = row0 + k * CH
            pltpu.sync_copy(acc_sh.at[pl.ds(r0, CH)],
                            out_hbm.at[pl.ds(cid * NP + r0, CH)])

    return pl.kernel(
        body,
        out_type=jax.ShapeDtypeStruct((NC * NP, D), jnp.float32),
        mesh=mesh,
        scratch_types=tuple(scratch))


def _make_deg_hist():
    mesh = plsc.VectorSubcoreMesh(
        core_axis_name="c", subcore_axis_name="s",
        num_cores=NC, num_subcores=NS)

    SEG = EPW // NSTR

    def body(dst_hbm, lanes_hbm, out_hbm, dst_all, lanes_v, hist):
        cid = lax.axis_index("c")
        sid = lax.axis_index("s")
        wid = sid * NC + cid

        pltpu.sync_copy(dst_hbm.at[pl.ds(wid * EPW, EPW)], dst_all)
        pltpu.sync_copy(lanes_hbm, lanes_v)
        lanes = lanes_v[...]

        def zero(i, _):
            for u in range(NSTR):
                hist[u, pl.ds(i * 16, 16)] = jnp.zeros((16,), jnp.float32)
            return 0
        lax.fori_loop(0, (NP + 16) // 16, zero, 0)

        def scan(g, _):
            for u in range(NSTR):
                vals = dst_all[pl.ds(u * SEG + g * 16, 16)]
                for j in range(16):
                    d = vals[j]
                    q16 = pl.multiple_of((d >> 4) * 16, 16)
                    oh = jnp.where(lanes == (d & 15), 1.0, 0.0)
                    hist[u, pl.ds(q16, 16)] = hist[u, pl.ds(q16, 16)] + oh
            return 0
        lax.fori_loop(0, SEG // 16, scan, 0)

        def merge(i, _):
            s = hist[0, pl.ds(i * 16, 16)]
            for u in range(1, NSTR):
                s = s + hist[u, pl.ds(i * 16, 16)]
            hist[0, pl.ds(i * 16, 16)] = s
            return 0
        lax.fori_loop(0, NP // 16, merge, 0)

        pltpu.sync_copy(hist.at[0, pl.ds(0, NP)], out_hbm.at[pl.ds(wid * NP, NP)])

    return pl.kernel(
        body,
        out_type=jax.ShapeDtypeStruct((NW * NP,), jnp.float32),
        mesh=mesh,
        scratch_types=(
            pltpu.VMEM((EPW,), jnp.int32),
            pltpu.VMEM((16,), jnp.int32),
            pltpu.VMEM((NSTR, NP + 16), jnp.float32),
        ))


_seg128 = _make_segsum(IN_DIM, stage_operand=False)
_deg_hist = _make_deg_hist()

_TCR = 1024
_NB = NP // _TCR


def _tc_layer1(x, pf, degh, wl1t, b1, wr1t, wr2t):
    def body(x_r, p0_r, p1_r, dg_r, wl1_r, b1_r, wr1_r, wr2_r, h_r, r2_r):
        deg = jnp.sum(dg_r[...], axis=0).reshape(_TCR, 1)
        invd = 1.0 / jnp.maximum(deg, 1.0)
        agg = (p0_r[...] + p1_r[...]) * invd
        out1 = (jnp.dot(agg, wl1_r[...], preferred_element_type=jnp.float32)
                + b1_r[...]
                + jnp.dot(x_r[...], wr1_r[...], preferred_element_type=jnp.float32))
        nrm = jnp.sqrt(jnp.sum(out1 * out1, axis=-1, keepdims=True))
        h = jnp.maximum(out1 / jnp.maximum(nrm, 1e-12), 0.0)
        h_r[...] = h
        r2_r[...] = jnp.dot(h, wr2_r[...], preferred_element_type=jnp.float32)

    row = lambda i: (i, 0)
    fix = lambda i: (0, 0)
    return pl.pallas_call(
        body,
        grid=(_NB,),
        in_specs=[
            pl.BlockSpec((_TCR, IN_DIM), row),
            pl.BlockSpec((_TCR, IN_DIM), row),
            pl.BlockSpec((_TCR, IN_DIM), lambda i: (i + _NB, 0)),
            pl.BlockSpec((NW, _TCR), lambda i: (0, i)),
            pl.BlockSpec((IN_DIM, HID), fix),
            pl.BlockSpec((1, HID), fix),
            pl.BlockSpec((IN_DIM, HID), fix),
            pl.BlockSpec((HID, OUT), fix),
        ],
        out_specs=[
            pl.BlockSpec((_TCR, HID), row),
            pl.BlockSpec((_TCR, OUT), row),
        ],
        out_shape=[
            jax.ShapeDtypeStruct((NP, HID), jnp.float32),
            jax.ShapeDtypeStruct((NP, OUT), jnp.float32),
        ],
    )(x, pf, pf, degh, wl1t, b1, wr1t, wr2t)


def _tc_layer2(qf, degh, r2, wl2t, b2):
    def body(q0_r, q1_r, dg_r, r2_r, wl2_r, b2_r, o_r):
        deg = jnp.sum(dg_r[...], axis=0).reshape(_TCR, 1)
        invd = 1.0 / jnp.maximum(deg, 1.0)
        agg = (q0_r[...] + q1_r[...]) * invd
        out2 = (jnp.dot(agg, wl2_r[...], preferred_element_type=jnp.float32)
                + b2_r[...] + r2_r[...])
        nrm = jnp.sqrt(jnp.sum(out2 * out2, axis=-1, keepdims=True))
        h2 = jnp.maximum(out2 / jnp.maximum(nrm, 1e-12), 0.0)
        m = jnp.max(h2, axis=-1, keepdims=True)
        e = jnp.exp(h2 - m)
        o_r[...] = (h2 - m) - jnp.log(jnp.sum(e, axis=-1, keepdims=True))

    row = lambda i: (i, 0)
    fix = lambda i: (0, 0)
    return pl.pallas_call(
        body,
        grid=(_NB,),
        in_specs=[
            pl.BlockSpec((_TCR, HID), row),
            pl.BlockSpec((_TCR, HID), lambda i: (i + _NB, 0)),
            pl.BlockSpec((NW, _TCR), lambda i: (0, i)),
            pl.BlockSpec((_TCR, OUT), row),
            pl.BlockSpec((HID, OUT), fix),
            pl.BlockSpec((1, OUT), fix),
        ],
        out_specs=pl.BlockSpec((_TCR, OUT), row),
        out_shape=jax.ShapeDtypeStruct((NP, OUT), jnp.float32),
    )(qf, qf, degh, r2, wl2t, b2)


def kernel(x, edge_index, W_l1, b_l1, W_r1, W_l2, b_l2, W_r2):
    x = x.astype(jnp.float32)
    src = edge_index[0].astype(jnp.int32)
    dst = edge_index[1].astype(jnp.int32)

    npad = EP - E
    spread = N + (jnp.arange(npad, dtype=jnp.int32) % (NP - N))
    src_p = jnp.concatenate([src, spread])
    dst_p = jnp.concatenate([dst, spread])
    x_p = jnp.pad(x, ((0, NP - N), (0, 0)))

    pf = _seg128(x_p, src_p, dst_p)
    degh = _deg_hist(dst_p, jnp.arange(16, dtype=jnp.int32))
    degh = degh.reshape(NW, NP)

    h, r2 = _tc_layer1(x_p, pf, degh,
                       W_l1.T, b_l1.reshape(1, HID), W_r1.T, W_r2.T)

    qf = _seg128(h, src_p, dst_p)

    out = _tc_layer2(qf, degh, r2, W_l2.T, b_l2.reshape(1, OUT))
    return out[:N]


# speedup vs baseline: 8.6542x; 1.4216x over previous
import jax
import jax.numpy as jnp
from jax import lax
from jax.experimental import pallas as pl
from jax.experimental.pallas import tpu as pltpu
from jax.experimental.pallas import tpu_sc as plsc

N = 10000
NP = 10240
E = 320000
EP = 327680
IN_DIM = 128
HID = 128
OUT = 64

NC = 2
NS = 16
NW = NC * NS
EPW = EP // NW
CH = 128
NCHUNK = EPW // CH
EPT = EPW + 2 * CH
RPT = NP // NS
NSTR = 8


def _zero_rows(ref, nrows, ncols):
    def body(i, _):
        for j in range(ncols // 16):
            ref[i, pl.ds(j * 16, 16)] = jnp.zeros((16,), jnp.float32)
        return 0
    lax.fori_loop(0, nrows, body, 0)


def _make_segsum(D, stage_operand):
    mesh = plsc.VectorSubcoreMesh(
        core_axis_name="c", subcore_axis_name="s",
        num_cores=NC, num_subcores=NS)

    scratch = [
        pltpu.VMEM((2, CH), jnp.int32),
        pltpu.VMEM((2, CH), jnp.int32),
        pltpu.VMEM((CH, D), jnp.float32),
        pltpu.VMEM((CH, D), jnp.float32),
        pltpu.VMEM_SHARED((NP, D), jnp.float32),
        pltpu.SemaphoreType.DMA,
        pltpu.SemaphoreType.DMA,
    ]

    def body(z_hbm, src_hbm, dst_hbm, out_hbm, *rest):
        src_v, dst_v, rows0_v, rows1_v, acc_sh, sem0, sem1 = rest
        cid = lax.axis_index("c")
        sid = lax.axis_index("s")
        wid = sid * NC + cid
        row0 = sid * RPT

        _zero_rows(rows0_v, CH, D)
        for k in range(RPT // CH):
            pltpu.sync_copy(rows0_v, acc_sh.at[pl.ds(row0 + k * CH, CH)])

        plsc.subcore_barrier()

        ebase = wid * EPT
        rows_b = (rows0_v, rows1_v)
        sem_b = (sem0, sem1)

        def fire(c, b):
            off = ebase + c * CH
            pltpu.sync_copy(src_hbm.at[pl.ds(off, CH)], src_v.at[b])
            pltpu.sync_copy(dst_hbm.at[pl.ds(off, CH)], dst_v.at[b])
            pltpu.async_copy(z_hbm.at[src_v.at[b]], rows_b[b], sem_b[b])

        fire(0, 0)
        fire(1, 1)

        def chunk(g, _):
            for b in range(2):
                pltpu.make_async_copy(z_hbm.at[src_v.at[b]],
                                      rows_b[b], sem_b[b]).wait()
                pltpu.sync_copy(rows_b[b], acc_sh.at[dst_v.at[b]], add=True)
                fire(2 * g + 2 + b, b)
            return 0
        lax.fori_loop(0, NCHUNK // 2, chunk, 0)

        for b in range(2):
            pltpu.make_async_copy(z_hbm.at[src_v.at[b]],
                                  rows_b[b], sem_b[b]).wait()

        plsc.subcore_barrier()

        for k in range(RPT // CH):
            r0 = row0 + k * CH
            pltpu.sync_copy(acc_sh.at[pl.ds(r0, CH)],
                            out_hbm.at[pl.ds(cid * NP + r0, CH)])

    return pl.kernel(
        body,
        out_type=jax.ShapeDtypeStruct((NC * NP, D), jnp.float32),
        mesh=mesh,
        scratch_types=tuple(scratch))


def _make_deg_hist():
    mesh = plsc.VectorSubcoreMesh(
        core_axis_name="c", subcore_axis_name="s",
        num_cores=NC, num_subcores=NS)

    SEG = EPT // NSTR

    def body(dst_hbm, lanes_hbm, out_hbm, dst_all, lanes_v, hist):
        cid = lax.axis_index("c")
        sid = lax.axis_index("s")
        wid = sid * NC + cid

        pltpu.sync_copy(dst_hbm.at[pl.ds(wid * EPT, EPT)], dst_all)
        pltpu.sync_copy(lanes_hbm, lanes_v)
        lanes = lanes_v[...]

        def zero(i, _):
            for u in range(NSTR):
                hist[u, pl.ds(i * 16, 16)] = jnp.zeros((16,), jnp.float32)
            return 0
        lax.fori_loop(0, (NP + 16) // 16, zero, 0)

        def scan(g, _):
            for u in range(NSTR):
                vals = dst_all[pl.ds(u * SEG + g * 16, 16)]
                for j in range(16):
                    d = vals[j]
                    q16 = pl.multiple_of((d >> 4) * 16, 16)
                    oh = jnp.where(lanes == (d & 15), 1.0, 0.0)
                    hist[u, pl.ds(q16, 16)] = hist[u, pl.ds(q16, 16)] + oh
            return 0
        lax.fori_loop(0, SEG // 16, scan, 0)

        def merge(i, _):
            s = hist[0, pl.ds(i * 16, 16)]
            for u in range(1, NSTR):
                s = s + hist[u, pl.ds(i * 16, 16)]
            hist[0, pl.ds(i * 16, 16)] = s
            return 0
        lax.fori_loop(0, NP // 16, merge, 0)

        pltpu.sync_copy(hist.at[0, pl.ds(0, NP)], out_hbm.at[pl.ds(wid * NP, NP)])

    return pl.kernel(
        body,
        out_type=jax.ShapeDtypeStruct((NW * NP,), jnp.float32),
        mesh=mesh,
        scratch_types=(
            pltpu.VMEM((EPT,), jnp.int32),
            pltpu.VMEM((16,), jnp.int32),
            pltpu.VMEM((NSTR, NP + 16), jnp.float32),
        ))


_seg128 = _make_segsum(IN_DIM, stage_operand=False)
_deg_hist = _make_deg_hist()

_TCR = 1024
_NB = NP // _TCR


def _tc_layer1(x, pf, degh, wl1t, b1, wr1t, wr2t):
    def body(x_r, p0_r, p1_r, dg_r, wl1_r, b1_r, wr1_r, wr2_r, h_r, r2_r):
        deg = jnp.sum(dg_r[...], axis=0).reshape(_TCR, 1)
        invd = 1.0 / jnp.maximum(deg, 1.0)
        agg = (p0_r[...] + p1_r[...]) * invd
        out1 = (jnp.dot(agg, wl1_r[...], preferred_element_type=jnp.float32)
                + b1_r[...]
                + jnp.dot(x_r[...], wr1_r[...], preferred_element_type=jnp.float32))
        nrm = jnp.sqrt(jnp.sum(out1 * out1, axis=-1, keepdims=True))
        h = jnp.maximum(out1 / jnp.maximum(nrm, 1e-12), 0.0)
        h_r[...] = h
        r2_r[...] = jnp.dot(h, wr2_r[...], preferred_element_type=jnp.float32)

    row = lambda i: (i, 0)
    fix = lambda i: (0, 0)
    return pl.pallas_call(
        body,
        grid=(_NB,),
        in_specs=[
            pl.BlockSpec((_TCR, IN_DIM), row),
            pl.BlockSpec((_TCR, IN_DIM), row),
            pl.BlockSpec((_TCR, IN_DIM), lambda i: (i + _NB, 0)),
            pl.BlockSpec((NW, _TCR), lambda i: (0, i)),
            pl.BlockSpec((IN_DIM, HID), fix),
            pl.BlockSpec((1, HID), fix),
            pl.BlockSpec((IN_DIM, HID), fix),
            pl.BlockSpec((HID, OUT), fix),
        ],
        out_specs=[
            pl.BlockSpec((_TCR, HID), row),
            pl.BlockSpec((_TCR, OUT), row),
        ],
        out_shape=[
            jax.ShapeDtypeStruct((NP, HID), jnp.float32),
            jax.ShapeDtypeStruct((NP, OUT), jnp.float32),
        ],
    )(x, pf, pf, degh, wl1t, b1, wr1t, wr2t)


def _tc_layer2(qf, degh, r2, wl2t, b2):
    def body(q0_r, q1_r, dg_r, r2_r, wl2_r, b2_r, o_r):
        deg = jnp.sum(dg_r[...], axis=0).reshape(_TCR, 1)
        invd = 1.0 / jnp.maximum(deg, 1.0)
        agg = (q0_r[...] + q1_r[...]) * invd
        out2 = (jnp.dot(agg, wl2_r[...], preferred_element_type=jnp.float32)
                + b2_r[...] + r2_r[...])
        nrm = jnp.sqrt(jnp.sum(out2 * out2, axis=-1, keepdims=True))
        h2 = jnp.maximum(out2 / jnp.maximum(nrm, 1e-12), 0.0)
        m = jnp.max(h2, axis=-1, keepdims=True)
        e = jnp.exp(h2 - m)
        o_r[...] = (h2 - m) - jnp.log(jnp.sum(e, axis=-1, keepdims=True))

    row = lambda i: (i, 0)
    fix = lambda i: (0, 0)
    return pl.pallas_call(
        body,
        grid=(_NB,),
        in_specs=[
            pl.BlockSpec((_TCR, HID), row),
            pl.BlockSpec((_TCR, HID), lambda i: (i + _NB, 0)),
            pl.BlockSpec((NW, _TCR), lambda i: (0, i)),
            pl.BlockSpec((_TCR, OUT), row),
            pl.BlockSpec((HID, OUT), fix),
            pl.BlockSpec((1, OUT), fix),
        ],
        out_specs=pl.BlockSpec((_TCR, OUT), row),
        out_shape=jax.ShapeDtypeStruct((NP, OUT), jnp.float32),
    )(qf, qf, degh, r2, wl2t, b2)


def kernel(x, edge_index, W_l1, b_l1, W_r1, W_l2, b_l2, W_r2):
    x = x.astype(jnp.float32)
    src = edge_index[0].astype(jnp.int32)
    dst = edge_index[1].astype(jnp.int32)

    npad = EP - E
    spread = N + (jnp.arange(npad, dtype=jnp.int32) % (NP - N))
    src_p = jnp.concatenate([src, spread]).reshape(NW, EPW)
    dst_p = jnp.concatenate([dst, spread]).reshape(NW, EPW)
    tail = N + (jnp.arange(NW * 2 * CH, dtype=jnp.int32)
                % (NP - N)).reshape(NW, 2 * CH)
    src_p = jnp.concatenate([src_p, tail], axis=1).reshape(NW * EPT)
    dst_p = jnp.concatenate([dst_p, tail], axis=1).reshape(NW * EPT)
    x_p = jnp.pad(x, ((0, NP - N), (0, 0)))

    pf = _seg128(x_p, src_p, dst_p)
    degh = _deg_hist(dst_p, jnp.arange(16, dtype=jnp.int32))
    degh = degh.reshape(NW, NP)

    h, r2 = _tc_layer1(x_p, pf, degh,
                       W_l1.T, b_l1.reshape(1, HID), W_r1.T, W_r2.T)

    qf = _seg128(h, src_p, dst_p)

    out = _tc_layer2(qf, degh, r2, W_l2.T, b_l2.reshape(1, OUT))
    return out[:N]


# speedup vs baseline: 9.7824x; 1.1304x over previous
import jax
import jax.numpy as jnp
from jax import lax
from jax.experimental import pallas as pl
from jax.experimental.pallas import tpu as pltpu
from jax.experimental.pallas import tpu_sc as plsc

N = 10000
NP = 10240
E = 320000
EP = 327680
IN_DIM = 128
HID = 128
OUT = 64

NC = 2
NS = 16
NW = NC * NS
EPW = EP // NW
CH = 128
NCHUNK = EPW // CH
NCHT = NCHUNK + 2
NCHP = 88
EPT = NCHP * CH
RPT = NP // NS
NSTR = 8


def _zero_rows(ref, nrows, ncols):
    def body(i, _):
        for j in range(ncols // 16):
            ref[i, pl.ds(j * 16, 16)] = jnp.zeros((16,), jnp.float32)
        return 0
    lax.fori_loop(0, nrows, body, 0)


def _make_segsum(D, stage_operand):
    mesh = plsc.VectorSubcoreMesh(
        core_axis_name="c", subcore_axis_name="s",
        num_cores=NC, num_subcores=NS)

    NIDX = NCHT * CH
    scratch = [
        pltpu.VMEM((2, CH), jnp.int32),
        pltpu.VMEM((NIDX,), jnp.int32),
        pltpu.VMEM((2, CH), jnp.int32),
        pltpu.VMEM((CH, D), jnp.float32),
        pltpu.VMEM((CH, D), jnp.float32),
        pltpu.VMEM_SHARED((NP, D), jnp.float32),
        pltpu.SemaphoreType.DMA,
        pltpu.SemaphoreType.DMA,
    ]

    def body(z_hbm, src_hbm, dst_hbm, out_hbm, *rest):
        src_v, dst_all, dst_v, rows0_v, rows1_v, acc_sh, sem0, sem1 = rest
        cid = lax.axis_index("c")
        sid = lax.axis_index("s")
        wid = sid * NC + cid
        row0 = sid * RPT

        pltpu.sync_copy(dst_hbm.at[pl.ds(wid * EPT, NIDX)], dst_all)
        _zero_rows(rows0_v, CH, D)
        for k in range(RPT // CH):
            pltpu.sync_copy(rows0_v, acc_sh.at[pl.ds(row0 + k * CH, CH)])

        plsc.subcore_barrier()

        rows_b = (rows0_v, rows1_v)
        sem_b = (sem0, sem1)
        ebase = wid * EPT

        def fire(c, b):
            pltpu.sync_copy(src_hbm.at[pl.ds(ebase + c * CH, CH)],
                            src_v.at[b])
            pltpu.async_copy(z_hbm.at[src_v.at[b]], rows_b[b], sem_b[b])
            for j in range(CH // 16):
                o16 = pl.multiple_of(c * CH + j * 16, 16)
                dst_v[b, pl.ds(j * 16, 16)] = dst_all[pl.ds(o16, 16)]

        fire(0, 0)
        fire(1, 1)

        def chunk(g, _):
            for b in range(2):
                c = 2 * g + b
                pltpu.make_async_copy(z_hbm.at[src_v.at[b]],
                                      rows_b[b], sem_b[b]).wait()
                pltpu.sync_copy(rows_b[b], acc_sh.at[dst_v.at[b]], add=True)
                fire(c + 2, b)
            return 0
        lax.fori_loop(0, NCHUNK // 2, chunk, 0)

        for b in range(2):
            pltpu.make_async_copy(z_hbm.at[src_v.at[b]],
                                  rows_b[b], sem_b[b]).wait()

        plsc.subcore_barrier()

        for k in range(RPT // CH):
            r0 = row0 + k * CH
            pltpu.sync_copy(acc_sh.at[pl.ds(r0, CH)],
                            out_hbm.at[pl.ds(cid * NP + r0, CH)])

    return pl.kernel(
        body,
        out_type=jax.ShapeDtypeStruct((NC * NP, D), jnp.float32),
        mesh=mesh,
        scratch_types=tuple(scratch))


def _make_deg_hist():
    mesh = plsc.VectorSubcoreMesh(
        core_axis_name="c", subcore_axis_name="s",
        num_cores=NC, num_subcores=NS)

    SEG = EPT // NSTR

    def body(dst_hbm, lanes_hbm, out_hbm, dst_all, lanes_v, hist):
        cid = lax.axis_index("c")
        sid = lax.axis_index("s")
        wid = sid * NC + cid

        pltpu.sync_copy(dst_hbm.at[pl.ds(wid * EPT, EPT)], dst_all)
        pltpu.sync_copy(lanes_hbm, lanes_v)
        lanes = lanes_v[...]

        def zero(i, _):
            for u in range(NSTR):
                hist[u, pl.ds(i * 16, 16)] = jnp.zeros((16,), jnp.float32)
            return 0
        lax.fori_loop(0, (NP + 16) // 16, zero, 0)

        def scan(g, _):
            for u in range(NSTR):
                vals = dst_all[pl.ds(u * SEG + g * 16, 16)]
                for j in range(16):
                    d = vals[j]
                    q16 = pl.multiple_of(d & ~15, 16)
                    oh = jnp.where(lanes == (d & 15), 1.0, 0.0)
                    hist[u, pl.ds(q16, 16)] = hist[u, pl.ds(q16, 16)] + oh
            return 0
        lax.fori_loop(0, SEG // 16, scan, 0)

        def merge(i, _):
            s = hist[0, pl.ds(i * 16, 16)]
            for u in range(1, NSTR):
                s = s + hist[u, pl.ds(i * 16, 16)]
            hist[0, pl.ds(i * 16, 16)] = s
            return 0
        lax.fori_loop(0, NP // 16, merge, 0)

        pltpu.sync_copy(hist.at[0, pl.ds(0, NP)], out_hbm.at[pl.ds(wid * NP, NP)])

    return pl.kernel(
        body,
        out_type=jax.ShapeDtypeStruct((NW * NP,), jnp.float32),
        mesh=mesh,
        scratch_types=(
            pltpu.VMEM((EPT,), jnp.int32),
            pltpu.VMEM((16,), jnp.int32),
            pltpu.VMEM((NSTR, NP + 16), jnp.float32),
        ))


_seg128 = _make_segsum(IN_DIM, stage_operand=False)
_deg_hist = _make_deg_hist()

_TCR = 1024
_NB = NP // _TCR


def _tc_layer1(x, pf, degh, wl1t, b1, wr1t, wr2t):
    def body(x_r, p0_r, p1_r, dg_r, wl1_r, b1_r, wr1_r, wr2_r, h_r, r2_r):
        deg = jnp.sum(dg_r[...], axis=0).reshape(_TCR, 1)
        invd = 1.0 / jnp.maximum(deg, 1.0)
        agg = (p0_r[...] + p1_r[...]) * invd
        out1 = (jnp.dot(agg, wl1_r[...], preferred_element_type=jnp.float32)
                + b1_r[...]
                + jnp.dot(x_r[...], wr1_r[...], preferred_element_type=jnp.float32))
        nrm = jnp.sqrt(jnp.sum(out1 * out1, axis=-1, keepdims=True))
        h = jnp.maximum(out1 / jnp.maximum(nrm, 1e-12), 0.0)
        h_r[...] = h
        r2_r[...] = jnp.dot(h, wr2_r[...], preferred_element_type=jnp.float32)

    row = lambda i: (i, 0)
    fix = lambda i: (0, 0)
    return pl.pallas_call(
        body,
        grid=(_NB,),
        in_specs=[
            pl.BlockSpec((_TCR, IN_DIM), row),
            pl.BlockSpec((_TCR, IN_DIM), row),
            pl.BlockSpec((_TCR, IN_DIM), lambda i: (i + _NB, 0)),
            pl.BlockSpec((NW, _TCR), lambda i: (0, i)),
            pl.BlockSpec((IN_DIM, HID), fix),
            pl.BlockSpec((1, HID), fix),
            pl.BlockSpec((IN_DIM, HID), fix),
            pl.BlockSpec((HID, OUT), fix),
        ],
        out_specs=[
            pl.BlockSpec((_TCR, HID), row),
            pl.BlockSpec((_TCR, OUT), row),
        ],
        out_shape=[
            jax.ShapeDtypeStruct((NP, HID), jnp.float32),
            jax.ShapeDtypeStruct((NP, OUT), jnp.float32),
        ],
    )(x, pf, pf, degh, wl1t, b1, wr1t, wr2t)


def _tc_layer2(qf, degh, r2, wl2t, b2):
    def body(q0_r, q1_r, dg_r, r2_r, wl2_r, b2_r, o_r):
        deg = jnp.sum(dg_r[...], axis=0).reshape(_TCR, 1)
        invd = 1.0 / jnp.maximum(deg, 1.0)
        agg = (q0_r[...] + q1_r[...]) * invd
        out2 = (jnp.dot(agg, wl2_r[...], preferred_element_type=jnp.float32)
                + b2_r[...] + r2_r[...])
        nrm = jnp.sqrt(jnp.sum(out2 * out2, axis=-1, keepdims=True))
        h2 = jnp.maximum(out2 / jnp.maximum(nrm, 1e-12), 0.0)
        m = jnp.max(h2, axis=-1, keepdims=True)
        e = jnp.exp(h2 - m)
        o_r[...] = (h2 - m) - jnp.log(jnp.sum(e, axis=-1, keepdims=True))

    row = lambda i: (i, 0)
    fix = lambda i: (0, 0)
    return pl.pallas_call(
        body,
        grid=(_NB,),
        in_specs=[
            pl.BlockSpec((_TCR, HID), row),
            pl.BlockSpec((_TCR, HID), lambda i: (i + _NB, 0)),
            pl.BlockSpec((NW, _TCR), lambda i: (0, i)),
            pl.BlockSpec((_TCR, OUT), row),
            pl.BlockSpec((HID, OUT), fix),
            pl.BlockSpec((1, OUT), fix),
        ],
        out_specs=pl.BlockSpec((_TCR, OUT), row),
        out_shape=jax.ShapeDtypeStruct((NP, OUT), jnp.float32),
    )(qf, qf, degh, r2, wl2t, b2)


def kernel(x, edge_index, W_l1, b_l1, W_r1, W_l2, b_l2, W_r2):
    x = x.astype(jnp.float32)
    src = edge_index[0].astype(jnp.int32)
    dst = edge_index[1].astype(jnp.int32)

    npad = EP - E
    spread = N + (jnp.arange(npad, dtype=jnp.int32) % (NP - N))
    src_p = jnp.concatenate([src, spread]).reshape(NW, EPW)
    dst_p = jnp.concatenate([dst, spread]).reshape(NW, EPW)
    tail = N + (jnp.arange(NW * (NCHP - NCHUNK) * CH, dtype=jnp.int32)
                % (NP - N)).reshape(NW, (NCHP - NCHUNK) * CH)
    src_c = jnp.concatenate([src_p, tail], axis=1).reshape(NW * EPT)
    dst_c = jnp.concatenate([dst_p, tail], axis=1).reshape(NW * EPT)
    dst_p = dst_c
    x_p = jnp.pad(x, ((0, NP - N), (0, 0)))

    pf = _seg128(x_p, src_c, dst_c)
    degh = _deg_hist(dst_p, jnp.arange(16, dtype=jnp.int32))
    degh = degh.reshape(NW, NP)

    h, r2 = _tc_layer1(x_p, pf, degh,
                       W_l1.T, b_l1.reshape(1, HID), W_r1.T, W_r2.T)

    qf = _seg128(h, src_c, dst_c)

    out = _tc_layer2(qf, degh, r2, W_l2.T, b_l2.reshape(1, OUT))
    return out[:N]


# speedup vs baseline: 10.5120x; 1.0746x over previous
import jax
import jax.numpy as jnp
from jax import lax
from jax.experimental import pallas as pl
from jax.experimental.pallas import tpu as pltpu
from jax.experimental.pallas import tpu_sc as plsc

N = 10000
NP = 10240
E = 320000
EP = 327680
IN_DIM = 128
HID = 128
OUT = 64

NC = 2
NS = 16
NW = NC * NS
EPW = EP // NW
CH = 128
NCHUNK = EPW // CH
NCHT = NCHUNK + 2
NCHP = 88
EPT = NCHP * CH
RPT = NP // NS
NSTR = 8


def _zero_rows(ref, nrows, ncols):
    def body(i, _):
        for j in range(ncols // 16):
            ref[i, pl.ds(j * 16, 16)] = jnp.zeros((16,), jnp.float32)
        return 0
    lax.fori_loop(0, nrows, body, 0)


def _make_segsum(D, stage_operand):
    mesh = plsc.VectorSubcoreMesh(
        core_axis_name="c", subcore_axis_name="s",
        num_cores=NC, num_subcores=NS)

    NIDX = NCHT * CH
    scratch = [
        pltpu.VMEM((2, 2, CH), jnp.int32),
        pltpu.VMEM((NIDX,), jnp.int32),
        pltpu.VMEM((2, CH), jnp.int32),
        pltpu.VMEM((CH, D), jnp.float32),
        pltpu.VMEM((CH, D), jnp.float32),
        pltpu.VMEM_SHARED((NP, D), jnp.float32),
        pltpu.SemaphoreType.DMA,
        pltpu.SemaphoreType.DMA,
        pltpu.SemaphoreType.DMA,
        pltpu.SemaphoreType.DMA,
        pltpu.SemaphoreType.DMA,
        pltpu.SemaphoreType.DMA,
    ]

    def body(z_hbm, src_hbm, dst_hbm, out_hbm, *rest):
        (src_v, dst_all, dst_v, rows0_v, rows1_v, acc_sh,
         sem0, sem1, si00, si01, si10, si11) = rest
        cid = lax.axis_index("c")
        sid = lax.axis_index("s")
        wid = sid * NC + cid
        row0 = sid * RPT

        pltpu.sync_copy(dst_hbm.at[pl.ds(wid * EPT, NIDX)], dst_all)
        _zero_rows(rows0_v, CH, D)
        for k in range(RPT // CH):
            pltpu.sync_copy(rows0_v, acc_sh.at[pl.ds(row0 + k * CH, CH)])

        plsc.subcore_barrier()

        rows_b = (rows0_v, rows1_v)
        sem_b = (sem0, sem1)
        sem_i = ((si00, si01), (si10, si11))
        ebase = wid * EPT

        def start_idx(c, b, s):
            pltpu.async_copy(src_hbm.at[pl.ds(ebase + c * CH, CH)],
                             src_v.at[b, s], sem_i[b][s])

        def wait_idx(b, s):
            pltpu.make_async_copy(src_hbm.at[pl.ds(ebase, CH)],
                                  src_v.at[b, s], sem_i[b][s]).wait()

        def fire_gather(b, s):
            pltpu.async_copy(z_hbm.at[src_v.at[b, s]], rows_b[b], sem_b[b])

        def wait_gather(b, s):
            pltpu.make_async_copy(z_hbm.at[src_v.at[b, s]],
                                  rows_b[b], sem_b[b]).wait()

        for b in range(2):
            start_idx(b, b, 0)
            start_idx(2 + b, b, 1)
        for b in range(2):
            wait_idx(b, 0)
            fire_gather(b, 0)

        def block(c, b, s):
            wait_gather(b, s)
            for j in range(CH // 16):
                o16 = pl.multiple_of(c * CH + j * 16, 16)
                dst_v[b, pl.ds(j * 16, 16)] = dst_all[pl.ds(o16, 16)]
            pltpu.sync_copy(rows_b[b], acc_sh.at[dst_v.at[b]], add=True)
            wait_idx(b, 1 - s)
            fire_gather(b, 1 - s)
            start_idx(c + 4, b, s)

        def round2(t, _):
            for gpar in range(2):
                g = 2 * t + gpar
                for b in range(2):
                    block(2 * g + b, b, gpar)
            return 0
        lax.fori_loop(0, NCHUNK // 4, round2, 0)

        for b in range(2):
            wait_gather(b, 0)
            wait_idx(b, 1)

        plsc.subcore_barrier()

        for k in range(RPT // CH):
            r0 = row0 + k * CH
            pltpu.sync_copy(acc_sh.at[pl.ds(r0, CH)],
                            out_hbm.at[pl.ds(cid * NP + r0, CH)])

    return pl.kernel(
        body,
        out_type=jax.ShapeDtypeStruct((NC * NP, D), jnp.float32),
        mesh=mesh,
        scratch_types=tuple(scratch))


def _make_deg_hist():
    mesh = plsc.VectorSubcoreMesh(
        core_axis_name="c", subcore_axis_name="s",
        num_cores=NC, num_subcores=NS)

    SEG = EPT // NSTR

    def body(dst_hbm, lanes_hbm, out_hbm, dst_all, lanes_v, hist):
        cid = lax.axis_index("c")
        sid = lax.axis_index("s")
        wid = sid * NC + cid

        pltpu.sync_copy(dst_hbm.at[pl.ds(wid * EPT, EPT)], dst_all)
        pltpu.sync_copy(lanes_hbm, lanes_v)
        lanes = lanes_v[...]

        def zero(i, _):
            for u in range(NSTR):
                hist[u, pl.ds(i * 16, 16)] = jnp.zeros((16,), jnp.float32)
            return 0
        lax.fori_loop(0, (NP + 16) // 16, zero, 0)

        def scan(g, _):
            for u in range(NSTR):
                vals = dst_all[pl.ds(u * SEG + g * 16, 16)]
                for j in range(16):
                    d = vals[j]
                    q16 = pl.multiple_of(d & ~15, 16)
                    oh = jnp.where(lanes == (d & 15), 1.0, 0.0)
                    hist[u, pl.ds(q16, 16)] = hist[u, pl.ds(q16, 16)] + oh
            return 0
        lax.fori_loop(0, SEG // 16, scan, 0)

        def merge(i, _):
            s = hist[0, pl.ds(i * 16, 16)]
            for u in range(1, NSTR):
                s = s + hist[u, pl.ds(i * 16, 16)]
            hist[0, pl.ds(i * 16, 16)] = s
            return 0
        lax.fori_loop(0, NP // 16, merge, 0)

        pltpu.sync_copy(hist.at[0, pl.ds(0, NP)], out_hbm.at[pl.ds(wid * NP, NP)])

    return pl.kernel(
        body,
        out_type=jax.ShapeDtypeStruct((NW * NP,), jnp.float32),
        mesh=mesh,
        scratch_types=(
            pltpu.VMEM((EPT,), jnp.int32),
            pltpu.VMEM((16,), jnp.int32),
            pltpu.VMEM((NSTR, NP + 16), jnp.float32),
        ))


_seg128 = _make_segsum(IN_DIM, stage_operand=False)
_deg_hist = _make_deg_hist()

_TCR = 1024
_NB = NP // _TCR


def _tc_layer1(x, pf, degh, wl1t, b1, wr1t, wr2t):
    def body(x_r, p0_r, p1_r, dg_r, wl1_r, b1_r, wr1_r, wr2_r, h_r, r2_r):
        deg = jnp.sum(dg_r[...], axis=0).reshape(_TCR, 1)
        invd = 1.0 / jnp.maximum(deg, 1.0)
        agg = (p0_r[...] + p1_r[...]) * invd
        out1 = (jnp.dot(agg, wl1_r[...], preferred_element_type=jnp.float32)
                + b1_r[...]
                + jnp.dot(x_r[...], wr1_r[...], preferred_element_type=jnp.float32))
        nrm = jnp.sqrt(jnp.sum(out1 * out1, axis=-1, keepdims=True))
        h = jnp.maximum(out1 / jnp.maximum(nrm, 1e-12), 0.0)
        h_r[...] = h
        r2_r[...] = jnp.dot(h, wr2_r[...], preferred_element_type=jnp.float32)

    row = lambda i: (i, 0)
    fix = lambda i: (0, 0)
    return pl.pallas_call(
        body,
        grid=(_NB,),
        in_specs=[
            pl.BlockSpec((_TCR, IN_DIM), row),
            pl.BlockSpec((_TCR, IN_DIM), row),
            pl.BlockSpec((_TCR, IN_DIM), lambda i: (i + _NB, 0)),
            pl.BlockSpec((NW, _TCR), lambda i: (0, i)),
            pl.BlockSpec((IN_DIM, HID), fix),
            pl.BlockSpec((1, HID), fix),
            pl.BlockSpec((IN_DIM, HID), fix),
            pl.BlockSpec((HID, OUT), fix),
        ],
        out_specs=[
            pl.BlockSpec((_TCR, HID), row),
            pl.BlockSpec((_TCR, OUT), row),
        ],
        out_shape=[
            jax.ShapeDtypeStruct((NP, HID), jnp.float32),
            jax.ShapeDtypeStruct((NP, OUT), jnp.float32),
        ],
    )(x, pf, pf, degh, wl1t, b1, wr1t, wr2t)


def _tc_layer2(qf, degh, r2, wl2t, b2):
    def body(q0_r, q1_r, dg_r, r2_r, wl2_r, b2_r, o_r):
        deg = jnp.sum(dg_r[...], axis=0).reshape(_TCR, 1)
        invd = 1.0 / jnp.maximum(deg, 1.0)
        agg = (q0_r[...] + q1_r[...]) * invd
        out2 = (jnp.dot(agg, wl2_r[...], preferred_element_type=jnp.float32)
                + b2_r[...] + r2_r[...])
        nrm = jnp.sqrt(jnp.sum(out2 * out2, axis=-1, keepdims=True))
        h2 = jnp.maximum(out2 / jnp.maximum(nrm, 1e-12), 0.0)
        m = jnp.max(h2, axis=-1, keepdims=True)
        e = jnp.exp(h2 - m)
        o_r[...] = (h2 - m) - jnp.log(jnp.sum(e, axis=-1, keepdims=True))

    row = lambda i: (i, 0)
    fix = lambda i: (0, 0)
    return pl.pallas_call(
        body,
        grid=(_NB,),
        in_specs=[
            pl.BlockSpec((_TCR, HID), row),
            pl.BlockSpec((_TCR, HID), lambda i: (i + _NB, 0)),
            pl.BlockSpec((NW, _TCR), lambda i: (0, i)),
            pl.BlockSpec((_TCR, OUT), row),
            pl.BlockSpec((HID, OUT), fix),
            pl.BlockSpec((1, OUT), fix),
        ],
        out_specs=pl.BlockSpec((_TCR, OUT), row),
        out_shape=jax.ShapeDtypeStruct((NP, OUT), jnp.float32),
    )(qf, qf, degh, r2, wl2t, b2)


def kernel(x, edge_index, W_l1, b_l1, W_r1, W_l2, b_l2, W_r2):
    x = x.astype(jnp.float32)
    src = edge_index[0].astype(jnp.int32)
    dst = edge_index[1].astype(jnp.int32)

    npad = EP - E
    spread = N + (jnp.arange(npad, dtype=jnp.int32) % (NP - N))
    src_p = jnp.concatenate([src, spread]).reshape(NW, EPW)
    dst_p = jnp.concatenate([dst, spread]).reshape(NW, EPW)
    tail = N + (jnp.arange(NW * (NCHP - NCHUNK) * CH, dtype=jnp.int32)
                % (NP - N)).reshape(NW, (NCHP - NCHUNK) * CH)
    src_c = jnp.concatenate([src_p, tail], axis=1).reshape(NW * EPT)
    dst_c = jnp.concatenate([dst_p, tail], axis=1).reshape(NW * EPT)
    dst_p = dst_c
    x_p = jnp.pad(x, ((0, NP - N), (0, 0)))

    pf = _seg128(x_p, src_c, dst_c)
    degh = _deg_hist(dst_p, jnp.arange(16, dtype=jnp.int32))
    degh = degh.reshape(NW, NP)

    h, r2 = _tc_layer1(x_p, pf, degh,
                       W_l1.T, b_l1.reshape(1, HID), W_r1.T, W_r2.T)

    qf = _seg128(h, src_c, dst_c)

    out = _tc_layer2(qf, degh, r2, W_l2.T, b_l2.reshape(1, OUT))
    return out[:N]


# speedup vs baseline: 10.5836x; 1.0068x over previous
import jax
import jax.numpy as jnp
from jax import lax
from jax.experimental import pallas as pl
from jax.experimental.pallas import tpu as pltpu
from jax.experimental.pallas import tpu_sc as plsc

N = 10000
NP = 10240
E = 320000
EP = 327680
IN_DIM = 128
HID = 128
OUT = 64

NC = 2
NS = 16
NW = NC * NS
EPW = EP // NW
CH = 128
NCHUNK = EPW // CH
NCHT = NCHUNK + 2
NCHP = 88
EPT = NCHP * CH
RPT = NP // NS
NSTR = 8


def _zero_rows(ref, nrows, ncols):
    def body(i, _):
        for j in range(ncols // 16):
            ref[i, pl.ds(j * 16, 16)] = jnp.zeros((16,), jnp.float32)
        return 0
    lax.fori_loop(0, nrows, body, 0)


def _make_segsum(D):
    mesh = plsc.VectorSubcoreMesh(
        core_axis_name="c", subcore_axis_name="s",
        num_cores=NC, num_subcores=NS)

    NIDX = NCHT * CH
    scratch = [
        pltpu.VMEM((2, 2, CH), jnp.int32),
        pltpu.VMEM((NIDX,), jnp.int32),
        pltpu.VMEM((2, CH), jnp.int32),
        pltpu.VMEM((CH, D), jnp.float32),
        pltpu.VMEM((CH, D), jnp.float32),
        pltpu.VMEM_SHARED((NP, D), jnp.float32),
        pltpu.SemaphoreType.DMA,
        pltpu.SemaphoreType.DMA,
        pltpu.SemaphoreType.DMA,
        pltpu.SemaphoreType.DMA,
        pltpu.SemaphoreType.DMA,
        pltpu.SemaphoreType.DMA,
    ]

    def body(z_hbm, src_hbm, dst_hbm, out_hbm, *rest):
        (src_v, dst_all, dst_v, rows0_v, rows1_v, acc_sh,
         sem0, sem1, si00, si01, si10, si11) = rest
        cid = lax.axis_index("c")
        sid = lax.axis_index("s")
        wid = sid * NC + cid
        row0 = sid * RPT

        pltpu.sync_copy(dst_hbm.at[pl.ds(wid * EPT, NIDX)], dst_all)
        _zero_rows(rows0_v, CH, D)
        for k in range(RPT // CH):
            pltpu.sync_copy(rows0_v, acc_sh.at[pl.ds(row0 + k * CH, CH)])

        plsc.subcore_barrier()

        rows_b = (rows0_v, rows1_v)
        sem_b = (sem0, sem1)
        sem_i = ((si00, si01), (si10, si11))
        ebase = wid * EPT

        def start_idx(c, b, s):
            pltpu.async_copy(src_hbm.at[pl.ds(ebase + c * CH, CH)],
                             src_v.at[b, s], sem_i[b][s])

        def wait_idx(b, s):
            pltpu.make_async_copy(src_hbm.at[pl.ds(ebase, CH)],
                                  src_v.at[b, s], sem_i[b][s]).wait()

        def fire_gather(b, s):
            pltpu.async_copy(z_hbm.at[src_v.at[b, s]], rows_b[b], sem_b[b])

        def wait_gather(b, s):
            pltpu.make_async_copy(z_hbm.at[src_v.at[b, s]],
                                  rows_b[b], sem_b[b]).wait()

        for b in range(2):
            start_idx(b, b, 0)
            start_idx(2 + b, b, 1)
        for b in range(2):
            wait_idx(b, 0)
            fire_gather(b, 0)

        def block(c, b, s):
            wait_gather(b, s)
            for j in range(CH // 16):
                o16 = pl.multiple_of(c * CH + j * 16, 16)
                dst_v[b, pl.ds(j * 16, 16)] = dst_all[pl.ds(o16, 16)]
            pltpu.sync_copy(rows_b[b], acc_sh.at[dst_v.at[b]], add=True)
            wait_idx(b, 1 - s)
            fire_gather(b, 1 - s)
            start_idx(c + 4, b, s)

        def round2(t, _):
            for gpar in range(2):
                g = 2 * t + gpar
                for b in range(2):
                    block(2 * g + b, b, gpar)
            return 0
        lax.fori_loop(0, NCHUNK // 4, round2, 0)

        for b in range(2):
            wait_gather(b, 0)
            wait_idx(b, 1)

        plsc.subcore_barrier()

        for k in range(RPT // CH):
            r0 = row0 + k * CH
            pltpu.sync_copy(acc_sh.at[pl.ds(r0, CH)],
                            out_hbm.at[pl.ds(cid * NP + r0, CH)])

    return pl.kernel(
        body,
        out_type=jax.ShapeDtypeStruct((NC * NP, D), jnp.float32),
        mesh=mesh,
        scratch_types=tuple(scratch))


def _make_deg_hist():
    mesh = plsc.VectorSubcoreMesh(
        core_axis_name="c", subcore_axis_name="s",
        num_cores=NC, num_subcores=NS)

    SEG = EPT // NSTR

    def body(dst_hbm, lanes_hbm, out_hbm, dst_all, lanes_v, hist):
        cid = lax.axis_index("c")
        sid = lax.axis_index("s")
        wid = sid * NC + cid

        pltpu.sync_copy(dst_hbm.at[pl.ds(wid * EPT, EPT)], dst_all)
        pltpu.sync_copy(lanes_hbm, lanes_v)
        lanes = lanes_v[...]

        def zero(i, _):
            for u in range(NSTR):
                hist[u, pl.ds(i * 16, 16)] = jnp.zeros((16,), jnp.float32)
            return 0
        lax.fori_loop(0, (NP + 16) // 16, zero, 0)

        def scan(g, _):
            for u in range(NSTR):
                vals = dst_all[pl.ds(u * SEG + g * 16, 16)]
                for j in range(16):
                    d = vals[j]
                    q16 = pl.multiple_of(d & ~15, 16)
                    oh = jnp.where(lanes == (d & 15), 1.0, 0.0)
                    hist[u, pl.ds(q16, 16)] = hist[u, pl.ds(q16, 16)] + oh
            return 0
        lax.fori_loop(0, SEG // 16, scan, 0)

        def merge(i, _):
            s = hist[0, pl.ds(i * 16, 16)]
            for u in range(1, NSTR):
                s = s + hist[u, pl.ds(i * 16, 16)]
            hist[0, pl.ds(i * 16, 16)] = s
            return 0
        lax.fori_loop(0, NP // 16, merge, 0)

        pltpu.sync_copy(hist.at[0, pl.ds(0, NP)], out_hbm.at[pl.ds(wid * NP, NP)])

    return pl.kernel(
        body,
        out_type=jax.ShapeDtypeStruct((NW * NP,), jnp.float32),
        mesh=mesh,
        scratch_types=(
            pltpu.VMEM((EPT,), jnp.int32),
            pltpu.VMEM((16,), jnp.int32),
            pltpu.VMEM((NSTR, NP + 16), jnp.float32),
        ))


_seg128 = _make_segsum(IN_DIM)
_deg_hist = _make_deg_hist()

_TCR = 1024
_NB = NP // _TCR


def _tc_layer1(x, pf, degh, wl1t, b1, wr1t, wr2t):
    def body(x_r, p0_r, p1_r, dg_r, wl1_r, b1_r, wr1_r, wr2_r, h_r, r2_r):
        deg = jnp.sum(dg_r[...], axis=0).reshape(_TCR, 1)
        invd = 1.0 / jnp.maximum(deg, 1.0)
        agg = (p0_r[...] + p1_r[...]) * invd
        out1 = (jnp.dot(agg, wl1_r[...], preferred_element_type=jnp.float32)
                + b1_r[...]
                + jnp.dot(x_r[...], wr1_r[...], preferred_element_type=jnp.float32))
        nrm = jnp.sqrt(jnp.sum(out1 * out1, axis=-1, keepdims=True))
        h = jnp.maximum(out1 / jnp.maximum(nrm, 1e-12), 0.0)
        h_r[...] = h
        r2_r[...] = jnp.dot(h, wr2_r[...], preferred_element_type=jnp.float32)

    row = lambda i: (i, 0)
    fix = lambda i: (0, 0)
    return pl.pallas_call(
        body,
        grid=(_NB,),
        in_specs=[
            pl.BlockSpec((_TCR, IN_DIM), row),
            pl.BlockSpec((_TCR, IN_DIM), row),
            pl.BlockSpec((_TCR, IN_DIM), lambda i: (i + _NB, 0)),
            pl.BlockSpec((NW, _TCR), lambda i: (0, i)),
            pl.BlockSpec((IN_DIM, HID), fix),
            pl.BlockSpec((1, HID), fix),
            pl.BlockSpec((IN_DIM, HID), fix),
            pl.BlockSpec((HID, OUT), fix),
        ],
        out_specs=[
            pl.BlockSpec((_TCR, HID), row),
            pl.BlockSpec((_TCR, OUT), row),
        ],
        out_shape=[
            jax.ShapeDtypeStruct((NP, HID), jnp.float32),
            jax.ShapeDtypeStruct((NP, OUT), jnp.float32),
        ],
    )(x, pf, pf, degh, wl1t, b1, wr1t, wr2t)


def _tc_layer2(qf, degh, r2, wl2t, b2):
    def body(q0_r, q1_r, dg_r, r2_r, wl2_r, b2_r, o_r):
        deg = jnp.sum(dg_r[...], axis=0).reshape(_TCR, 1)
        invd = 1.0 / jnp.maximum(deg, 1.0)
        agg = (q0_r[...] + q1_r[...]) * invd
        out2 = (jnp.dot(agg, wl2_r[...], preferred_element_type=jnp.float32)
                + b2_r[...] + r2_r[...])
        nrm = jnp.sqrt(jnp.sum(out2 * out2, axis=-1, keepdims=True))
        h2 = jnp.maximum(out2 / jnp.maximum(nrm, 1e-12), 0.0)
        m = jnp.max(h2, axis=-1, keepdims=True)
        e = jnp.exp(h2 - m)
        o_r[...] = (h2 - m) - jnp.log(jnp.sum(e, axis=-1, keepdims=True))

    row = lambda i: (i, 0)
    fix = lambda i: (0, 0)
    return pl.pallas_call(
        body,
        grid=(_NB,),
        in_specs=[
            pl.BlockSpec((_TCR, HID), row),
            pl.BlockSpec((_TCR, HID), lambda i: (i + _NB, 0)),
            pl.BlockSpec((NW, _TCR), lambda i: (0, i)),
            pl.BlockSpec((_TCR, OUT), row),
            pl.BlockSpec((HID, OUT), fix),
            pl.BlockSpec((1, OUT), fix),
        ],
        out_specs=pl.BlockSpec((_TCR, OUT), row),
        out_shape=jax.ShapeDtypeStruct((NP, OUT), jnp.float32),
    )(qf, qf, degh, r2, wl2t, b2)


def kernel(x, edge_index, W_l1, b_l1, W_r1, W_l2, b_l2, W_r2):
    x = x.astype(jnp.float32)
    src = edge_index[0].astype(jnp.int32)
    dst = edge_index[1].astype(jnp.int32)

    npad = EP - E
    spread = N + (jnp.arange(npad, dtype=jnp.int32) % (NP - N))
    src_p = jnp.concatenate([src, spread]).reshape(NW, EPW)
    dst_p = jnp.concatenate([dst, spread]).reshape(NW, EPW)
    tail = N + (jnp.arange(NW * (NCHP - NCHUNK) * CH, dtype=jnp.int32)
                % (NP - N)).reshape(NW, (NCHP - NCHUNK) * CH)
    src_c = jnp.concatenate([src_p, tail], axis=1).reshape(NW * EPT)
    dst_c = jnp.concatenate([dst_p, tail], axis=1).reshape(NW * EPT)
    dst_p = dst_c
    x_p = jnp.pad(x, ((0, NP - N), (0, 0)))

    pf = _seg128(x_p, src_c, dst_c)
    degh = _deg_hist(dst_p, jnp.arange(16, dtype=jnp.int32))
    degh = degh.reshape(NW, NP)

    h, r2 = _tc_layer1(x_p, pf, degh,
                       W_l1.T, b_l1.reshape(1, HID), W_r1.T, W_r2.T)

    qf = _seg128(h, src_c, dst_c)

    out = _tc_layer2(qf, degh, r2, W_l2.T, b_l2.reshape(1, OUT))
    return out[:N]
